# baseline (speedup 1.0000x reference)
import functools

import jax
import jax.numpy as jnp
from jax import lax
from jax.experimental import pallas as pl
from jax.experimental.pallas import tpu as pltpu
from jax.experimental.pallas import tpu_sc as plsc

_B = 1024
_K = 100000
_NW = 32
_RPW = _B // _NW
_R = 16
_W = 2000
_NC = _K // _W
_NG = _RPW // _R


def _sc_body(state_hbm, values_hbm, out_hbm, sbuf, vbuf, outv):
    wid = lax.axis_index("s") * 2 + lax.axis_index("c")
    rb = wid * _RPW
    lanes = lax.iota(jnp.int32, 16)

    for g in range(_NG):
        row0 = rb + g * _R

        def chunk_body(c, accs, row0=row0):
            pltpu.sync_copy(values_hbm.at[pl.ds(c * _W, _W)], vbuf)
            pltpu.sync_copy(
                state_hbm.at[pl.ds(row0, _R), pl.ds(c * _W, _W)], sbuf)

            def j_body(j, accs):
                v = vbuf[pl.ds(j * 16, 16)]
                return tuple(
                    accs[i] + sbuf[i, pl.ds(j * 16, 16)] * v
                    for i in range(_R))

            return lax.fori_loop(0, _W // 16, j_body, accs)

        accs0 = tuple(jnp.zeros((16,), jnp.float32) for _ in range(_R))
        accs = lax.fori_loop(0, _NC, chunk_body, accs0)

        out_vec = jnp.zeros((16,), jnp.float32)
        for i in range(_R):
            out_vec = jnp.where(lanes == i, jnp.sum(accs[i]), out_vec)
        outv[pl.ds(g * 16, 16)] = out_vec

    pltpu.sync_copy(outv, out_hbm.at[pl.ds(rb, _RPW)])


_sc_matvec = functools.partial(
    pl.kernel,
    out_type=jax.ShapeDtypeStruct((_B,), jnp.float32),
    mesh=plsc.VectorSubcoreMesh(core_axis_name="c", subcore_axis_name="s"),
    scratch_types=[
        pltpu.VMEM((_R, _W), jnp.float32),
        pltpu.VMEM((_W,), jnp.float32),
        pltpu.VMEM((_RPW,), jnp.float32),
    ],
    compiler_params=pltpu.CompilerParams(
        use_tc_tiling_on_sc=False, needs_layout_passes=False),
)(_sc_body)


def kernel(state, values):
    out = _sc_matvec(state, values.reshape(_K))
    return out.reshape(_B, 1)


# speedup vs baseline: 1.1927x; 1.1927x over previous
import functools

import jax
import jax.numpy as jnp
from jax import lax
from jax.experimental import pallas as pl
from jax.experimental.pallas import tpu as pltpu
from jax.experimental.pallas import tpu_sc as plsc

_B = 1024
_K = 100000
_NW = 32
_RPW = _B // _NW
_R = 16
_W = 2000
_NC = _K // _W
_NG = _RPW // _R


def _sc_body(state_hbm, values_hbm, out_hbm, sbuf, vbuf, outv,
             ssem0, ssem1, vsem0, vsem1):
    wid = lax.axis_index("s") * 2 + lax.axis_index("c")
    rb = wid * _RPW
    lanes = lax.iota(jnp.int32, 16)
    ssems = (ssem0, ssem1)
    vsems = (vsem0, vsem1)

    for g in range(_NG):
        row0 = rb + g * _R

        def start(c, slot, row0=row0):
            pltpu.async_copy(
                state_hbm.at[pl.ds(row0, _R), pl.ds(c * _W, _W)],
                sbuf.at[slot], ssems[slot])
            pltpu.async_copy(
                values_hbm.at[pl.ds(c * _W, _W)], vbuf.at[slot], vsems[slot])

        def wait(c, slot, row0=row0):
            pltpu.make_async_copy(
                state_hbm.at[pl.ds(row0, _R), pl.ds(c * _W, _W)],
                sbuf.at[slot], ssems[slot]).wait()
            pltpu.make_async_copy(
                values_hbm.at[pl.ds(c * _W, _W)], vbuf.at[slot],
                vsems[slot]).wait()

        def compute(slot, accs):
            def j_body(j, accs):
                v = vbuf[slot, pl.ds(j * 16, 16)]
                return tuple(
                    accs[i] + sbuf[slot, i, pl.ds(j * 16, 16)] * v
                    for i in range(_R))
            return plsc.parallel_loop(
                0, _W // 16, 1, unroll=2, carry=accs)(j_body)

        start(0, 0)
        start(1, 1)

        def pair_body(p, accs):
            for b in range(2):
                c = 2 * p + b
                wait(c, b)
                accs = compute(b, accs)

                @pl.when(c + 2 < _NC)
                def _(c=c, b=b):
                    start(c + 2, b)
            return accs

        accs0 = tuple(jnp.zeros((16,), jnp.float32) for _ in range(_R))
        accs = lax.fori_loop(0, _NC // 2, pair_body, accs0)

        out_vec = jnp.zeros((16,), jnp.float32)
        for i in range(_R):
            out_vec = jnp.where(lanes == i, jnp.sum(accs[i]), out_vec)
        outv[pl.ds(g * 16, 16)] = out_vec

    pltpu.sync_copy(outv, out_hbm.at[pl.ds(rb, _RPW)])


_sc_matvec = functools.partial(
    pl.kernel,
    out_type=jax.ShapeDtypeStruct((_B,), jnp.float32),
    mesh=plsc.VectorSubcoreMesh(core_axis_name="c", subcore_axis_name="s"),
    scratch_types=[
        pltpu.VMEM((2, _R, _W), jnp.float32),
        pltpu.VMEM((2, _W), jnp.float32),
        pltpu.VMEM((_RPW,), jnp.float32),
        pltpu.SemaphoreType.DMA,
        pltpu.SemaphoreType.DMA,
        pltpu.SemaphoreType.DMA,
        pltpu.SemaphoreType.DMA,
    ],
    compiler_params=pltpu.CompilerParams(
        use_tc_tiling_on_sc=False, needs_layout_passes=False),
)(_sc_body)


def kernel(state, values):
    out = _sc_matvec(state, values.reshape(_K))
    return out.reshape(_B, 1)


# speedup vs baseline: 1.5345x; 1.2865x over previous
import functools

import jax
import jax.numpy as jnp
from jax import lax
from jax.experimental import pallas as pl
from jax.experimental.pallas import tpu as pltpu

_B = 1024
_K = 100000
_BM = 8


def _tc_body(v_ref, s_ref, o_ref):
    o_ref[...] = jnp.dot(s_ref[...], v_ref[...],
                         preferred_element_type=jnp.float32)


_tc_matvec = pl.pallas_call(
    _tc_body,
    grid=(_B // _BM,),
    in_specs=[
        pl.BlockSpec((_K, 1), lambda c: (0, 0)),
        pl.BlockSpec((_BM, _K), lambda c: (c, 0)),
    ],
    out_specs=pl.BlockSpec((_BM, 1), lambda c: (c, 0)),
    out_shape=jax.ShapeDtypeStruct((_B, 1), jnp.float32),
)


def kernel(state, values):
    return _tc_matvec(values, state)


# speedup vs baseline: 2.3482x; 1.5303x over previous
import functools

import jax
import jax.numpy as jnp
from jax import lax
from jax.experimental import pallas as pl
from jax.experimental.pallas import tpu as pltpu

_B = 1024
_K = 100000
_BM = 8


def _tc_body(v_ref, s_ref, o_ref):
    prod = s_ref[...] * v_ref[...]
    o_ref[...] = jnp.sum(prod, axis=1, keepdims=True)


_tc_matvec = pl.pallas_call(
    _tc_body,
    grid=(_B // _BM,),
    in_specs=[
        pl.BlockSpec((1, _K), lambda c: (0, 0)),
        pl.BlockSpec((_BM, _K), lambda c: (c, 0)),
    ],
    out_specs=pl.BlockSpec((_BM, 1), lambda c: (c, 0)),
    out_shape=jax.ShapeDtypeStruct((_B, 1), jnp.float32),
)


def kernel(state, values):
    return _tc_matvec(values.reshape(1, _K), state)


# speedup vs baseline: 2.6628x; 1.1339x over previous
import functools

import jax
import jax.numpy as jnp
from jax import lax
from jax.experimental import pallas as pl
from jax.experimental.pallas import tpu as pltpu

_B = 1024
_K = 100000
_BM = 32


def _tc_body(v_ref, s_ref, o_ref):
    prod = s_ref[...] * v_ref[...]
    o_ref[...] = jnp.sum(prod, axis=1, keepdims=True)


_tc_matvec = pl.pallas_call(
    _tc_body,
    grid=(_B // _BM,),
    in_specs=[
        pl.BlockSpec((1, _K), lambda c: (0, 0)),
        pl.BlockSpec((_BM, _K), lambda c: (c, 0)),
    ],
    out_specs=pl.BlockSpec((_BM, 1), lambda c: (c, 0)),
    out_shape=jax.ShapeDtypeStruct((_B, 1), jnp.float32),
)


def kernel(state, values):
    return _tc_matvec(values.reshape(1, _K), state)
